# baseline (speedup 1.0000x reference)
import jax
import jax.numpy as jnp
import numpy as np
from jax.experimental import pallas as pl
from jax.experimental.pallas import tpu as pltpu

_IDX = np.array(
    [3, 17, 29, 42, 56, 61, 73, 88, 91, 104, 111, 123, 130, 142, 150,
     158, 163, 171, 180, 187, 195, 7, 12, 25, 33, 47, 52, 66, 79, 83,
     96, 101, 115, 127, 135, 146, 153, 167, 174, 182, 190, 199, 5, 19,
     38, 59, 70, 99, 119, 139],
    dtype=np.int32,
)
_K = _IDX.shape[0]


def _body(idx_ref, x_ref, o_ref, sem):
    copies = [
        pltpu.make_async_copy(
            x_ref.at[pl.ds(idx_ref[j], 1)], o_ref.at[pl.ds(j, 1)], sem
        )
        for j in range(_K)
    ]
    for c in copies:
        c.start()
    for c in copies:
        c.wait()


def kernel(x):
    B, R, F = x.shape
    xt = jnp.transpose(x, (1, 2, 0))
    idx = jnp.asarray(_IDX)

    out_t = pl.pallas_call(
        _body,
        grid_spec=pltpu.PrefetchScalarGridSpec(
            num_scalar_prefetch=1,
            grid=(),
            in_specs=[pl.BlockSpec(memory_space=pltpu.MemorySpace.HBM)],
            out_specs=pl.BlockSpec(memory_space=pltpu.MemorySpace.HBM),
            scratch_shapes=[pltpu.SemaphoreType.DMA],
        ),
        out_shape=jax.ShapeDtypeStruct((_K, F, B), x.dtype),
    )(idx, xt)
    return out_t.transpose(2, 0, 1)


# speedup vs baseline: 34.6026x; 34.6026x over previous
import functools
import jax
import jax.numpy as jnp
import numpy as np
from jax import lax
from jax.experimental import pallas as pl
from jax.experimental.pallas import tpu as pltpu
from jax.experimental.pallas import tpu_sc as plsc

_IDX = np.array(
    [3, 17, 29, 42, 56, 61, 73, 88, 91, 104, 111, 123, 130, 142, 150,
     158, 163, 171, 180, 187, 195, 7, 12, 25, 33, 47, 52, 66, 79, 83,
     96, 101, 115, 127, 135, 146, 153, 167, 174, 182, 190, 199, 5, 19,
     38, 59, 70, 99, 119, 139],
    dtype=np.int32,
)

_B, _R, _F = 16384, 200, 64
_K = _IDX.shape[0]
_NW = 32
_NROW = _K * _F
_ROWS_PER_W = _NROW // _NW
_CHUNK = 4
_NCHUNK = _ROWS_PER_W // _CHUNK

_ROW_IDX = (_IDX[:, None] * _F
            + np.arange(_F, dtype=np.int32)[None, :]).reshape(-1, _CHUNK)


def kernel(x):
    xt = jnp.transpose(x, (1, 2, 0))
    x4 = xt.reshape(_R * _F, _B)
    idx = jnp.asarray(_ROW_IDX)
    mesh = plsc.VectorSubcoreMesh(core_axis_name="c", subcore_axis_name="s")

    @functools.partial(
        pl.kernel,
        mesh=mesh,
        out_type=jax.ShapeDtypeStruct((_NROW, _B), jnp.float32),
        scratch_types=[
            pltpu.VMEM((_CHUNK,), jnp.int32),
            pltpu.VMEM((_CHUNK, _B), jnp.float32),
            pltpu.SemaphoreType.DMA,
        ],
        compiler_params=pltpu.CompilerParams(use_tc_tiling_on_sc=True),
    )
    def sc_gather(x_hbm, idx_hbm, out_hbm, idx_v, rows_v, sem):
        wid = lax.axis_index("s") * 2 + lax.axis_index("c")
        base_w = wid * _ROWS_PER_W

        def chunk_body(i, carry):
            base = base_w + i * _CHUNK
            chunk_row = wid * _NCHUNK + i
            pltpu.sync_copy(idx_hbm.at[chunk_row], idx_v)
            pltpu.async_copy(x_hbm.at[idx_v], rows_v, sem).wait()
            pltpu.sync_copy(rows_v, out_hbm.at[pl.ds(base, _CHUNK)])
            return carry

        lax.fori_loop(0, _NCHUNK, chunk_body, 0)

    out4 = sc_gather(x4, idx)
    return out4.reshape(_K, _F, _B).transpose(2, 0, 1)


# speedup vs baseline: 37.8496x; 1.0938x over previous
import functools
import jax
import jax.numpy as jnp
import numpy as np
from jax import lax
from jax.experimental import pallas as pl
from jax.experimental.pallas import tpu as pltpu
from jax.experimental.pallas import tpu_sc as plsc

_IDX = np.array(
    [3, 17, 29, 42, 56, 61, 73, 88, 91, 104, 111, 123, 130, 142, 150,
     158, 163, 171, 180, 187, 195, 7, 12, 25, 33, 47, 52, 66, 79, 83,
     96, 101, 115, 127, 135, 146, 153, 167, 174, 182, 190, 199, 5, 19,
     38, 59, 70, 99, 119, 139],
    dtype=np.int32,
)

_B, _R, _F = 16384, 200, 64
_K = _IDX.shape[0]
_NW = 32
_NROW = _K * _F
_ROWS_PER_W = _NROW // _NW
_CHUNK = 2
_NCHUNK = _ROWS_PER_W // _CHUNK
_NBUF = 2

_ROW_IDX = (_IDX[:, None] * _F
            + np.arange(_F, dtype=np.int32)[None, :]
            ).reshape(_NW, _NCHUNK, _CHUNK)


def kernel(x):
    xt = jnp.transpose(x, (1, 2, 0))
    x4 = xt.reshape(_R * _F, _B)
    idx = jnp.asarray(_ROW_IDX)
    mesh = plsc.VectorSubcoreMesh(core_axis_name="c", subcore_axis_name="s")

    @functools.partial(
        pl.kernel,
        mesh=mesh,
        out_type=jax.ShapeDtypeStruct((_NROW, _B), jnp.float32),
        scratch_types=[
            pltpu.VMEM((_NCHUNK, _CHUNK), jnp.int32),
            pltpu.VMEM((_NBUF, _CHUNK, _B), jnp.float32),
            pltpu.SemaphoreType.DMA((_NBUF,)),
            pltpu.SemaphoreType.DMA((_NBUF,)),
        ],
        compiler_params=pltpu.CompilerParams(use_tc_tiling_on_sc=True),
    )
    def sc_gather(x_hbm, idx_hbm, out_hbm, idx_v, rows_v, gsem, wsem):
        wid = lax.axis_index("s") * 2 + lax.axis_index("c")
        base_w = wid * _ROWS_PER_W

        pltpu.sync_copy(idx_hbm.at[wid], idx_v)

        def gather(g, b):
            return pltpu.make_async_copy(
                x_hbm.at[idx_v.at[g]], rows_v.at[b], gsem.at[b]
            )

        def write(g, b):
            return pltpu.make_async_copy(
                rows_v.at[b], out_hbm.at[pl.ds(base_w + g * _CHUNK, _CHUNK)],
                wsem.at[b],
            )

        for b in range(_NBUF):
            gather(b, b).start()

        def round_body(k, carry):
            for b in range(_NBUF):
                g = k * _NBUF + b
                gather(g, b).wait()
                write(g, b).start()
                write(g, b).wait()
                nxt = g + _NBUF

                @pl.when(nxt < _NCHUNK)
                def _():
                    gather(nxt, b).start()

            return carry

        lax.fori_loop(0, _NCHUNK // _NBUF, round_body, 0)

    out4 = sc_gather(x4, idx)
    return out4.reshape(_K, _F, _B).transpose(2, 0, 1)


# speedup vs baseline: 37.9258x; 1.0020x over previous
import functools
import jax
import jax.numpy as jnp
import numpy as np
from jax import lax
from jax.experimental import pallas as pl
from jax.experimental.pallas import tpu as pltpu
from jax.experimental.pallas import tpu_sc as plsc

_IDX = np.array(
    [3, 17, 29, 42, 56, 61, 73, 88, 91, 104, 111, 123, 130, 142, 150,
     158, 163, 171, 180, 187, 195, 7, 12, 25, 33, 47, 52, 66, 79, 83,
     96, 101, 115, 127, 135, 146, 153, 167, 174, 182, 190, 199, 5, 19,
     38, 59, 70, 99, 119, 139],
    dtype=np.int32,
)

_B, _R, _F = 16384, 200, 64
_K = _IDX.shape[0]
_NW = 32
_NROW = _K * _F
_ROWS_PER_W = _NROW // _NW
_CHUNK = 2
_NCHUNK = _ROWS_PER_W // _CHUNK
_NBUF = 3

_ROW_IDX = (_IDX[:, None] * _F
            + np.arange(_F, dtype=np.int32)[None, :]
            ).reshape(_NW, _NCHUNK, _CHUNK)

assert _NCHUNK % _NBUF != 0 or True


def kernel(x):
    xt = jnp.transpose(x, (1, 2, 0))
    x4 = xt.reshape(_R * _F, _B)
    idx = jnp.asarray(_ROW_IDX)
    mesh = plsc.VectorSubcoreMesh(core_axis_name="c", subcore_axis_name="s")

    @functools.partial(
        pl.kernel,
        mesh=mesh,
        out_type=jax.ShapeDtypeStruct((_NROW, _B), jnp.float32),
        scratch_types=[
            pltpu.VMEM((_NCHUNK, _CHUNK), jnp.int32),
            pltpu.VMEM((_NBUF, _CHUNK, _B), jnp.float32),
            pltpu.SemaphoreType.DMA((_NBUF,)),
            pltpu.SemaphoreType.DMA((_NBUF,)),
        ],
        compiler_params=pltpu.CompilerParams(use_tc_tiling_on_sc=True),
    )
    def sc_gather(x_hbm, idx_hbm, out_hbm, idx_v, rows_v, gsem, wsem):
        wid = lax.axis_index("s") * 2 + lax.axis_index("c")
        base_w = wid * _ROWS_PER_W

        pltpu.sync_copy(idx_hbm.at[wid], idx_v)

        def gather(g, b):
            return pltpu.make_async_copy(
                x_hbm.at[idx_v.at[g]], rows_v.at[b], gsem.at[b]
            )

        def write(g, b):
            return pltpu.make_async_copy(
                rows_v.at[b], out_hbm.at[pl.ds(base_w + g * _CHUNK, _CHUNK)],
                wsem.at[b],
            )

        gather(0, 0).start()
        gather(1, 1).start()

        def round_body(k, carry):
            for b in range(_NBUF):
                g = k * _NBUF + b
                gather(g, b).wait()
                write(g, b).start()
                nb = (b + 2) % _NBUF
                nxt = g + 2

                @pl.when(nxt < _NCHUNK)
                def _():
                    @pl.when(g >= 1)
                    def _():
                        write(g - 1, nb).wait()

                    gather(nxt, nb).start()

            return carry

        nrounds = _NCHUNK // _NBUF
        lax.fori_loop(0, nrounds, round_body, 0)

        for g in range(nrounds * _NBUF, _NCHUNK):
            b = g % _NBUF
            gather(g, b).wait()
            write(g - 1, (g + 2) % _NBUF).wait()
            write(g, b).start()

        write(_NCHUNK - 1, (_NCHUNK - 1) % _NBUF).wait()

    out4 = sc_gather(x4, idx)
    return out4.reshape(_K, _F, _B).transpose(2, 0, 1)


# speedup vs baseline: 49.0058x; 1.2921x over previous
import jax
import jax.numpy as jnp
import numpy as np
from jax.experimental import pallas as pl
from jax.experimental.pallas import tpu as pltpu

_IDX = [3, 17, 29, 42, 56, 61, 73, 88, 91, 104, 111, 123, 130, 142, 150,
        158, 163, 171, 180, 187, 195, 7, 12, 25, 33, 47, 52, 66, 79, 83,
        96, 101, 115, 127, 135, 146, 153, 167, 174, 182, 190, 199, 5, 19,
        38, 59, 70, 99, 119, 139]

_B, _R, _F = 16384, 200, 64
_K = len(_IDX)
_NBUF = 6
_LOOKAHEAD = 4


def _body(x_ref, o_ref, buf, gsem, wsem):
    def gather(j, b):
        return pltpu.make_async_copy(
            x_ref.at[pl.ds(_IDX[j], 1)], buf.at[b], gsem.at[b]
        )

    def write(j, b):
        return pltpu.make_async_copy(
            buf.at[b], o_ref.at[pl.ds(j, 1)], wsem.at[b]
        )

    for j in range(_LOOKAHEAD):
        gather(j, j % _NBUF).start()

    for j in range(_K):
        b = j % _NBUF
        gather(j, b).wait()
        write(j, b).start()
        nxt = j + _LOOKAHEAD
        if nxt < _K:
            prev = nxt - _NBUF
            if prev >= 0:
                write(prev, nxt % _NBUF).wait()
            gather(nxt, nxt % _NBUF).start()

    for j in range(max(0, _K - _NBUF), _K):
        write(j, j % _NBUF).wait()


def kernel(x):
    B, R, F = x.shape
    xt = jnp.transpose(x, (1, 2, 0))

    out_t = pl.pallas_call(
        _body,
        grid=(),
        in_specs=[pl.BlockSpec(memory_space=pltpu.MemorySpace.HBM)],
        out_specs=pl.BlockSpec(memory_space=pltpu.MemorySpace.HBM),
        scratch_shapes=[
            pltpu.VMEM((_NBUF, 1, F, B), jnp.float32),
            pltpu.SemaphoreType.DMA((_NBUF,)),
            pltpu.SemaphoreType.DMA((_NBUF,)),
        ],
        out_shape=jax.ShapeDtypeStruct((_K, F, B), x.dtype),
    )(xt)
    return out_t.transpose(2, 0, 1)
